# Initial kernel scaffold; baseline (speedup 1.0000x reference)
import jax
import jax.numpy as jnp
from jax.experimental import pallas as pl


def kernel(x, pos, batch):
    raise NotImplementedError("write your pallas kernel here")


# speedup vs baseline: 1.6314x; 1.6314x over previous
import functools

import jax
import jax.numpy as jnp
from jax import lax
from jax.experimental import pallas as pl
from jax.experimental.pallas import tpu as pltpu
from jax.experimental.pallas import tpu_sc as plsc

_N = 100000
_D = 128
_DIM = 1024
_PAD = 100352
_ROWS = 784

_BLK = 400
_NBLK = _N // _BLK
_BPW = -(-_NBLK // 16)

_NEG = float("-inf")


def _cluster_body(p0_ref, p1_ref, b_ref, cid_ref):
    p0 = p0_ref[...]
    p1 = p1_ref[...]
    b = b_ref[...]
    sx = jnp.min(p0)
    ex = jnp.max(p0)
    sy = jnp.min(p1)
    ey = jnp.max(p1)
    k0 = jnp.floor((p0 - sx) / 0.125).astype(jnp.int32)
    k1 = jnp.floor((p1 - sy) / 0.125).astype(jnp.int32)
    num0 = jnp.floor((ex - sx) / 0.125).astype(jnp.int32) + 1
    num1 = jnp.floor((ey - sy) / 0.125).astype(jnp.int32) + 1
    bmin = jnp.min(b)
    cid_ref[...] = k0 + k1 * num0 + (b - bmin) * (num0 * num1)


def _compute_cluster(pos, batch):
    p0 = jnp.pad(pos[:, 0], (0, _PAD - _N), mode="edge").reshape(_ROWS, _D)
    p1 = jnp.pad(pos[:, 1], (0, _PAD - _N), mode="edge").reshape(_ROWS, _D)
    b = jnp.pad(batch, (0, _PAD - _N), mode="edge").reshape(_ROWS, _D)
    cid = pl.pallas_call(
        _cluster_body,
        out_shape=jax.ShapeDtypeStruct((_ROWS, _D), jnp.int32),
    )(p0, p1, b)
    return cid.reshape(-1)[:_N]


def _segmax_body(x_hbm, cid_hbm, out_hbm, xblk, cidv, acc, parts, res,
                 tmp, sem):
    c = lax.axis_index("c")
    s = lax.axis_index("s")
    fbase = c * 64

    neg16 = jnp.full((16,), _NEG, dtype=jnp.float32)

    def init_row(i, carry):
        acc[pl.ds(i * 16, 16)] = neg16
        return carry

    lax.fori_loop(0, _DIM * 4, init_row, 0)

    def block(k, carry):
        bid = s + k * 16

        @pl.when(bid < _NBLK)
        def _():
            base = bid * _BLK
            pltpu.sync_copy(cid_hbm.at[pl.ds(base, _BLK)], cidv)
            pltpu.sync_copy(x_hbm.at[pl.ds(base, _BLK), :], xblk)

            def group(g, carry2):
                cvec = cidv[pl.ds(g * 16, 16)]
                for j in range(16):
                    cc = cvec[j]
                    i = g * 16 + j
                    for f in range(4):
                        asl = pl.ds(cc * 64 + f * 16, 16)
                        xsl = pl.ds(fbase + f * 16, 16)
                        acc[asl] = jnp.maximum(acc[asl], xblk[i, xsl])
                return carry2

            lax.fori_loop(0, _BLK // 16, group, 0)

        return carry

    lax.fori_loop(0, _BPW, block, 0)

    zeros16 = jnp.zeros((16,), jnp.float32)
    wid = c * 16 + s
    pltpu.sync_copy(acc, parts.at[pl.ds(wid * 65536, 65536)])
    plsc.subcore_barrier()

    wbase = s * 4096
    pltpu.sync_copy(parts.at[pl.ds(c * 16 * 65536 + wbase, 4096)], res)

    def merge(q, carry):
        pltpu.sync_copy(
            parts.at[pl.ds((c * 16 + q) * 65536 + wbase, 4096)], tmp)

        def chunk(i, carry2):
            sl = pl.ds(i * 16, 16)
            res[sl] = jnp.maximum(res[sl], tmp[sl])
            return carry2

        lax.fori_loop(0, 256, chunk, 0)
        return carry

    lax.fori_loop(1, 16, merge, 0)

    def finalize(i, carry):
        sl = pl.ds(i * 16, 16)
        v = res[sl]
        res[sl] = jnp.where(v == _NEG, zeros16, v)
        return carry

    lax.fori_loop(0, 256, finalize, 0)
    pltpu.sync_copy(res, out_hbm.at[c, pl.ds(wbase, 4096)])


_segmax = functools.partial(
    pl.kernel,
    out_type=jax.ShapeDtypeStruct((2, _DIM * 64), jnp.float32),
    mesh=plsc.VectorSubcoreMesh(core_axis_name="c", subcore_axis_name="s"),
    scratch_types=[
        pltpu.VMEM((_BLK, _D), jnp.float32),
        pltpu.VMEM((_BLK,), jnp.int32),
        pltpu.VMEM((_DIM * 64,), jnp.float32),
        pltpu.HBM((32 * _DIM * 64,), jnp.float32),
        pltpu.VMEM((4096,), jnp.float32),
        pltpu.VMEM((4096,), jnp.float32),
        pltpu.SemaphoreType.DMA,
    ],
)(_segmax_body)


def kernel(x, pos, batch):
    cid = _compute_cluster(pos, batch)
    halves = _segmax(x, cid)
    halves = halves.reshape(2, _DIM, 64)
    return jnp.concatenate([halves[0], halves[1]], axis=1)
